# Initial kernel scaffold; baseline (speedup 1.0000x reference)
import jax
import jax.numpy as jnp
from jax.experimental import pallas as pl


def kernel(x_pfc, edge_index, W_lin, W_src, W_dst, W_pos, b_pos):
    raise NotImplementedError("write your pallas kernel here")


# speedup vs baseline: 15.9410x; 15.9410x over previous
import functools

import jax
import jax.numpy as jnp
from jax import lax
from jax.experimental import pallas as pl
from jax.experimental.pallas import tpu as pltpu
from jax.experimental.pallas import tpu_sc as plsc

NC = 2
NS = 16
NW = NC * NS
CH = 128
BLK = 1024


def _tc_tables(xt_aug, wl_t, ws_t, wd_t, wp_t, n_pad):
  nblk = n_pad // BLK

  def body(x_ref, wl_ref, ws_ref, wd_ref, wp_ref,
           u0_r, u1_r, p0_r, p1_r, v0_r, v1_r, w0_r, w1_r, exs_r, ts_r):
    xt = x_ref[...]
    adt = jnp.dot(wd_ref[...], xt, preferred_element_type=jnp.float32, precision=lax.Precision.HIGHEST)
    avt = jnp.dot(ws_ref[...], xt, preferred_element_type=jnp.float32, precision=lax.Precision.HIGHEST)
    xvt = jnp.dot(wl_ref[...], xt, preferred_element_type=jnp.float32, precision=lax.Precision.HIGHEST)
    pt = jnp.dot(wp_ref[...], xt[:4, :], preferred_element_type=jnp.float32, precision=lax.Precision.HIGHEST)
    u0_r[...] = adt[0] + pt[0]
    u1_r[...] = adt[1] + pt[1]
    p0_r[...] = pt[0]
    p1_r[...] = pt[1]
    v0_r[...] = avt[0] + pt[0]
    v1_r[...] = avt[1] + pt[1]
    w0_r[...] = xvt[0] - pt[0]
    w1_r[...] = xvt[1] - pt[1]
    exs_r[...] = jnp.exp(adt - avt)
    ts_r[...] = xvt

  full = lambda s: pl.BlockSpec(s, lambda i: (0, 0))
  vec = pl.BlockSpec((BLK,), lambda i: (i,))
  row2 = pl.BlockSpec((2, BLK), lambda i: (0, i))
  vshape = jax.ShapeDtypeStruct((n_pad,), jnp.float32)
  return pl.pallas_call(
      body,
      grid=(nblk,),
      in_specs=[
          pl.BlockSpec((16, BLK), lambda i: (0, i)),
          full((2, 16)), full((2, 16)), full((2, 16)), full((2, 4)),
      ],
      out_specs=[vec] * 8 + [row2, row2],
      out_shape=[vshape] * 8 + [
          jax.ShapeDtypeStruct((2, n_pad), jnp.float32),
          jax.ShapeDtypeStruct((2, n_pad), jnp.float32),
      ],
  )(xt_aug, wl_t, ws_t, wd_t, wp_t)


def _tc_dinv(dpart, exs_t, ts_t, n_pad):
  nblk = n_pad // BLK

  def body(dp_ref, exs_ref, ts_ref, d0_ref, d1_ref, self_ref):
    dp = dp_ref[...]
    exs = exs_ref[...]
    dinv = 1.0 / (dp[0] + dp[1] + exs + 1e-16)
    d0_ref[...] = dinv[0]
    d1_ref[...] = dinv[1]
    self_ref[...] = exs * dinv * ts_ref[...]

  vec = pl.BlockSpec((BLK,), lambda i: (i,))
  row2 = pl.BlockSpec((2, BLK), lambda i: (0, i))
  return pl.pallas_call(
      body,
      grid=(nblk,),
      in_specs=[
          pl.BlockSpec((2, 2, BLK), lambda i: (0, 0, i)),
          row2, row2,
      ],
      out_specs=[vec, vec, row2],
      out_shape=[
          jax.ShapeDtypeStruct((n_pad,), jnp.float32),
          jax.ShapeDtypeStruct((n_pad,), jnp.float32),
          jax.ShapeDtypeStruct((2, n_pad), jnp.float32),
      ],
  )(dpart, exs_t, ts_t)


def _tc_final(opart, selfmsg, n_pad):
  nblk = n_pad // BLK

  def body(op_ref, sm_ref, out_ref):
    op = op_ref[...]
    out_ref[...] = op[0] + op[1] + sm_ref[...]

  row2 = pl.BlockSpec((2, BLK), lambda i: (0, i))
  return pl.pallas_call(
      body,
      grid=(nblk,),
      in_specs=[
          pl.BlockSpec((2, 2, BLK), lambda i: (0, 0, i)),
          row2,
      ],
      out_specs=row2,
      out_shape=jax.ShapeDtypeStruct((2, n_pad), jnp.float32),
  )(opart, selfmsg)


def _sc_pass_a(eif, u0, u1, p0, p1, v0, v1, w0, w1, n_pad, e, trash):
  nch_total = e // CH
  base, rem = nch_total // NW, nch_total % NW
  mesh = plsc.VectorSubcoreMesh(core_axis_name="c", subcore_axis_name="s")
  rps = n_pad // NS

  @functools.partial(
      pl.kernel,
      out_type=(
          jax.ShapeDtypeStruct((e,), jnp.float32),
          jax.ShapeDtypeStruct((e,), jnp.float32),
          jax.ShapeDtypeStruct((e,), jnp.int32),
          jax.ShapeDtypeStruct((NC, 2, n_pad), jnp.float32),
      ),
      mesh=mesh,
      compiler_params=pltpu.CompilerParams(needs_layout_passes=False),
      scratch_types=[
          pltpu.VMEM((CH,), jnp.int32),
          pltpu.VMEM((CH,), jnp.int32),
          pltpu.VMEM((CH,), jnp.int32),
          [pltpu.VMEM((CH,), jnp.float32) for _ in range(8)],
          pltpu.VMEM((CH,), jnp.float32),
          pltpu.VMEM((CH,), jnp.float32),
          pltpu.VMEM((CH,), jnp.float32),
          pltpu.VMEM((CH,), jnp.float32),
          pltpu.VMEM((rps,), jnp.float32),
          [pltpu.VMEM_SHARED((n_pad,), jnp.float32) for _ in range(2)],
          pltpu.SemaphoreType.DMA,
      ],
  )
  def k(eif_ref, u0_ref, u1_ref, p0_ref, p1_ref, v0_ref, v1_ref, w0_ref,
        w1_ref, num0_o, num1_o, md_o, dpart_o,
        dstb, srcb, mdb, gbufs, ex0b, ex1b, num0b, num1b, stage, dens, sem):
    cid = lax.axis_index("c")
    sid = lax.axis_index("s")
    wid = sid * NC + cid
    r0 = pl.multiple_of(sid * rps, 8)
    for g in range(rps // 16):
      stage[pl.ds(g * 16, 16)] = jnp.zeros((16,), jnp.float32)
    pltpu.sync_copy(stage, dens[0].at[pl.ds(r0, rps)])
    pltpu.sync_copy(stage, dens[1].at[pl.ds(r0, rps)])
    plsc.subcore_barrier()

    tabs = (u0_ref, u1_ref, p0_ref, p1_ref, v0_ref, v1_ref, w0_ref, w1_ref)

    def body(i, carry):
      ch = wid + i * NW
      e0 = pl.multiple_of(ch * CH, CH)
      pltpu.sync_copy(eif_ref.at[pl.ds(e0, CH)], dstb)
      pltpu.sync_copy(eif_ref.at[pl.ds(e + e0, CH)], srcb)
      cps = []
      for t in range(8):
        idx = dstb if t < 4 else srcb
        cps.append(pltpu.async_copy(tabs[t].at[idx], gbufs[t], sem))
      for cp in cps:
        cp.wait()
      for g in range(CH // 16):
        sl = pl.ds(g * 16, 16)
        dv = dstb[sl]
        sv = srcb[sl]
        mdb[sl] = jnp.where(dv == sv, trash, dv)
        a0 = jnp.exp(gbufs[0][sl] - gbufs[4][sl])
        a1 = jnp.exp(gbufs[1][sl] - gbufs[5][sl])
        ex0b[sl] = a0
        ex1b[sl] = a1
        num0b[sl] = a0 * (gbufs[2][sl] + gbufs[6][sl])
        num1b[sl] = a1 * (gbufs[3][sl] + gbufs[7][sl])
      pltpu.sync_copy(ex0b, dens[0].at[mdb], add=True)
      pltpu.sync_copy(ex1b, dens[1].at[mdb], add=True)
      pltpu.sync_copy(num0b, num0_o.at[pl.ds(e0, CH)])
      pltpu.sync_copy(num1b, num1_o.at[pl.ds(e0, CH)])
      pltpu.sync_copy(mdb, md_o.at[pl.ds(e0, CH)])
      return carry

    nch = jnp.where(wid < rem, base + 1, base)
    lax.fori_loop(0, nch, body, 0)
    plsc.subcore_barrier()
    for c in range(2):
      pltpu.sync_copy(dens[c].at[pl.ds(r0, rps)], stage)
      pltpu.sync_copy(stage, dpart_o.at[cid, c, pl.ds(r0, rps)])

  return k(eif, u0, u1, p0, p1, v0, v1, w0, w1)


def _sc_pass_b(num0, num1, md, d0, d1, n_pad, e):
  nch_total = e // CH
  base, rem = nch_total // NW, nch_total % NW
  mesh = plsc.VectorSubcoreMesh(core_axis_name="c", subcore_axis_name="s")
  rps = n_pad // NS

  @functools.partial(
      pl.kernel,
      out_type=jax.ShapeDtypeStruct((NC, 2, n_pad), jnp.float32),
      mesh=mesh,
      compiler_params=pltpu.CompilerParams(needs_layout_passes=False),
      scratch_types=[
          pltpu.VMEM((CH,), jnp.int32),
          pltpu.VMEM((CH,), jnp.float32),
          pltpu.VMEM((CH,), jnp.float32),
          pltpu.VMEM((CH,), jnp.float32),
          pltpu.VMEM((CH,), jnp.float32),
          pltpu.VMEM((CH,), jnp.float32),
          pltpu.VMEM((CH,), jnp.float32),
          pltpu.VMEM((rps,), jnp.float32),
          [pltpu.VMEM_SHARED((n_pad,), jnp.float32) for _ in range(2)],
          pltpu.SemaphoreType.DMA,
      ],
  )
  def k(num0_ref, num1_ref, md_ref, d0_ref, d1_ref, opart_o,
        mdb, num0b, num1b, g0b, g1b, m0b, m1b, stage, outs, sem):
    cid = lax.axis_index("c")
    sid = lax.axis_index("s")
    wid = sid * NC + cid
    r0 = pl.multiple_of(sid * rps, 8)
    for g in range(rps // 16):
      stage[pl.ds(g * 16, 16)] = jnp.zeros((16,), jnp.float32)
    pltpu.sync_copy(stage, outs[0].at[pl.ds(r0, rps)])
    pltpu.sync_copy(stage, outs[1].at[pl.ds(r0, rps)])
    plsc.subcore_barrier()

    def body(i, carry):
      ch = wid + i * NW
      e0 = pl.multiple_of(ch * CH, CH)
      pltpu.sync_copy(md_ref.at[pl.ds(e0, CH)], mdb)
      pltpu.sync_copy(num0_ref.at[pl.ds(e0, CH)], num0b)
      pltpu.sync_copy(num1_ref.at[pl.ds(e0, CH)], num1b)
      cp1 = pltpu.async_copy(d0_ref.at[mdb], g0b, sem)
      cp2 = pltpu.async_copy(d1_ref.at[mdb], g1b, sem)
      cp1.wait()
      cp2.wait()
      for g in range(CH // 16):
        sl = pl.ds(g * 16, 16)
        m0b[sl] = num0b[sl] * g0b[sl]
        m1b[sl] = num1b[sl] * g1b[sl]
      pltpu.sync_copy(m0b, outs[0].at[mdb], add=True)
      pltpu.sync_copy(m1b, outs[1].at[mdb], add=True)
      return carry

    nch = jnp.where(wid < rem, base + 1, base)
    lax.fori_loop(0, nch, body, 0)
    plsc.subcore_barrier()
    for c in range(2):
      pltpu.sync_copy(outs[c].at[pl.ds(r0, rps)], stage)
      pltpu.sync_copy(stage, opart_o.at[cid, c, pl.ds(r0, rps)])

  return k(num0, num1, md, d0, d1)


def kernel(x_pfc, edge_index, W_lin, W_src, W_dst, W_pos, b_pos):
  n = x_pfc.shape[0]
  e = edge_index.shape[1]
  n_pad = ((n + 1 + BLK - 1) // BLK) * BLK
  trash = jnp.int32(n)

  x_pad = jnp.pad(x_pfc, ((0, n_pad - n), (0, 0)))
  xt_aug = jnp.concatenate(
      [x_pad.T, jnp.ones((1, n_pad), jnp.float32)], axis=0)
  b2 = b_pos.reshape(2, 1)
  z2 = jnp.zeros((2, 1), jnp.float32)
  wd_t = jnp.concatenate([W_dst.T, b2], axis=1)
  ws_t = jnp.concatenate([W_src.T, z2], axis=1)
  wl_t = jnp.concatenate([W_lin.T, b2], axis=1)
  wp_t = W_pos.T
  eif = edge_index.reshape(-1)

  u0, u1, p0, p1, v0, v1, w0, w1, exs_t, ts_t = _tc_tables(
      xt_aug, wl_t, ws_t, wd_t, wp_t, n_pad)
  num0, num1, md, dpart = _sc_pass_a(
      eif, u0, u1, p0, p1, v0, v1, w0, w1, n_pad, e, trash)
  d0, d1, selfmsg = _tc_dinv(dpart, exs_t, ts_t, n_pad)
  opart = _sc_pass_b(num0, num1, md, d0, d1, n_pad, e)
  out_t = _tc_final(opart, selfmsg, n_pad)
  return out_t.T[:n]


# speedup vs baseline: 23.5795x; 1.4792x over previous
import functools

import jax
import jax.numpy as jnp
from jax import lax
from jax.experimental import pallas as pl
from jax.experimental.pallas import tpu as pltpu
from jax.experimental.pallas import tpu_sc as plsc

NC = 2
NS = 16
NW = NC * NS
CH = 128
BLK = 1024


def _tc_tables(xt_aug, wl_t, ws_t, wd_t, wp_t, n_pad):
  nblk = n_pad // BLK

  def body(x_ref, wl_ref, ws_ref, wd_ref, wp_ref,
           u0_r, u1_r, p0_r, p1_r, v0_r, v1_r, w0_r, w1_r, exs_r, ts_r):
    xt = x_ref[...]
    adt = jnp.dot(wd_ref[...], xt, preferred_element_type=jnp.float32, precision=lax.Precision.HIGHEST)
    avt = jnp.dot(ws_ref[...], xt, preferred_element_type=jnp.float32, precision=lax.Precision.HIGHEST)
    xvt = jnp.dot(wl_ref[...], xt, preferred_element_type=jnp.float32, precision=lax.Precision.HIGHEST)
    pt = jnp.dot(wp_ref[...], xt[:4, :], preferred_element_type=jnp.float32, precision=lax.Precision.HIGHEST)
    u0_r[...] = adt[0] + pt[0]
    u1_r[...] = adt[1] + pt[1]
    p0_r[...] = pt[0]
    p1_r[...] = pt[1]
    v0_r[...] = avt[0] + pt[0]
    v1_r[...] = avt[1] + pt[1]
    w0_r[...] = xvt[0] - pt[0]
    w1_r[...] = xvt[1] - pt[1]
    exs_r[...] = jnp.exp(adt - avt)
    ts_r[...] = xvt

  full = lambda s: pl.BlockSpec(s, lambda i: (0, 0))
  vec = pl.BlockSpec((BLK,), lambda i: (i,))
  row2 = pl.BlockSpec((2, BLK), lambda i: (0, i))
  vshape = jax.ShapeDtypeStruct((n_pad,), jnp.float32)
  return pl.pallas_call(
      body,
      grid=(nblk,),
      in_specs=[
          pl.BlockSpec((16, BLK), lambda i: (0, i)),
          full((2, 16)), full((2, 16)), full((2, 16)), full((2, 4)),
      ],
      out_specs=[vec] * 8 + [row2, row2],
      out_shape=[vshape] * 8 + [
          jax.ShapeDtypeStruct((2, n_pad), jnp.float32),
          jax.ShapeDtypeStruct((2, n_pad), jnp.float32),
      ],
  )(xt_aug, wl_t, ws_t, wd_t, wp_t)


def _tc_dinv(dpart, exs_t, ts_t, n_pad):
  nblk = n_pad // BLK

  def body(dp_ref, exs_ref, ts_ref, d0_ref, d1_ref, self_ref):
    dp = dp_ref[...]
    exs = exs_ref[...]
    dinv = 1.0 / (dp[0] + dp[1] + exs + 1e-16)
    d0_ref[...] = dinv[0]
    d1_ref[...] = dinv[1]
    self_ref[...] = exs * dinv * ts_ref[...]

  vec = pl.BlockSpec((BLK,), lambda i: (i,))
  row2 = pl.BlockSpec((2, BLK), lambda i: (0, i))
  return pl.pallas_call(
      body,
      grid=(nblk,),
      in_specs=[
          pl.BlockSpec((2, 2, BLK), lambda i: (0, 0, i)),
          row2, row2,
      ],
      out_specs=[vec, vec, row2],
      out_shape=[
          jax.ShapeDtypeStruct((n_pad,), jnp.float32),
          jax.ShapeDtypeStruct((n_pad,), jnp.float32),
          jax.ShapeDtypeStruct((2, n_pad), jnp.float32),
      ],
  )(dpart, exs_t, ts_t)


def _tc_final(opart, selfmsg, n_pad):
  nblk = n_pad // BLK

  def body(op_ref, sm_ref, out_ref):
    op = op_ref[...]
    out_ref[...] = op[0] + op[1] + sm_ref[...]

  row2 = pl.BlockSpec((2, BLK), lambda i: (0, i))
  return pl.pallas_call(
      body,
      grid=(nblk,),
      in_specs=[
          pl.BlockSpec((2, 2, BLK), lambda i: (0, 0, i)),
          row2,
      ],
      out_specs=row2,
      out_shape=jax.ShapeDtypeStruct((2, n_pad), jnp.float32),
  )(opart, selfmsg)


def _sc_pass_a(epk, u0, u1, p0, p1, v0, v1, w0, w1, n_pad, e, trash):
  nch_total = e // CH
  base, rem = nch_total // NW, nch_total % NW
  mesh = plsc.VectorSubcoreMesh(core_axis_name="c", subcore_axis_name="s")
  rps = n_pad // NS

  @functools.partial(
      pl.kernel,
      out_type=(
          jax.ShapeDtypeStruct((3 * e,), jnp.float32),
          jax.ShapeDtypeStruct((NC, 2, n_pad), jnp.float32),
      ),
      mesh=mesh,
      compiler_params=pltpu.CompilerParams(needs_layout_passes=False),
      scratch_types=[
          [pltpu.VMEM((2 * CH,), jnp.int32) for _ in range(2)],
          [[pltpu.VMEM((CH,), jnp.float32) for _ in range(8)]
           for _ in range(2)],
          pltpu.VMEM((CH,), jnp.int32),
          pltpu.VMEM((CH,), jnp.float32),
          pltpu.VMEM((CH,), jnp.float32),
          pltpu.VMEM((3 * CH,), jnp.float32),
          pltpu.VMEM((rps,), jnp.float32),
          [pltpu.VMEM_SHARED((n_pad,), jnp.float32) for _ in range(2)],
          pltpu.SemaphoreType.DMA,
          pltpu.SemaphoreType.DMA,
          pltpu.SemaphoreType.DMA,
      ],
  )
  def k(epk_ref, u0_ref, u1_ref, p0_ref, p1_ref, v0_ref, v1_ref, w0_ref,
        w1_ref, pk_o, dpart_o,
        idxbs, gbs, mdb, ex0b, ex1b, pkb, stage, dens, sem_i, sem_g, sem_o):
    cid = lax.axis_index("c")
    sid = lax.axis_index("s")
    wid = sid * NC + cid
    r0 = pl.multiple_of(sid * rps, 8)
    for g in range(rps // 16):
      stage[pl.ds(g * 16, 16)] = jnp.zeros((16,), jnp.float32)
    pltpu.sync_copy(stage, dens[0].at[pl.ds(r0, rps)])
    pltpu.sync_copy(stage, dens[1].at[pl.ds(r0, rps)])
    plsc.subcore_barrier()

    tabs = (u0_ref, u1_ref, p0_ref, p1_ref, v0_ref, v1_ref, w0_ref, w1_ref)
    nch = jnp.where(wid < rem, base + 1, base)

    def chunk_of(i):
      return wid + i * NW

    def issue_idx(i, bank):
      i0 = pl.multiple_of(chunk_of(i) * (2 * CH), 2 * CH)
      return pltpu.async_copy(epk_ref.at[pl.ds(i0, 2 * CH)], idxbs[bank],
                              sem_i)

    def issue_gathers(bank):
      for t in range(8):
        sl = pl.ds(0, CH) if t < 4 else pl.ds(CH, CH)
        pltpu.async_copy(tabs[t].at[idxbs[bank].at[sl]], gbs[bank][t], sem_g)

    def wait_gathers(bank):
      for t in range(8):
        sl = pl.ds(0, CH) if t < 4 else pl.ds(CH, CH)
        pltpu.make_async_copy(tabs[t].at[idxbs[bank].at[sl]], gbs[bank][t],
                              sem_g).wait()

    def issue_outs(i):
      e0p = pl.multiple_of(chunk_of(i) * (3 * CH), 3 * CH)
      pltpu.sync_copy(ex0b, dens[0].at[mdb], add=True)
      pltpu.sync_copy(ex1b, dens[1].at[mdb], add=True)
      pltpu.sync_copy(pkb, pk_o.at[pl.ds(e0p, 3 * CH)])

    def compute(bank):
      idxb = idxbs[bank]
      gb = gbs[bank]
      for g in range(CH // 16):
        sl = pl.ds(g * 16, 16)
        dv = idxb[sl]
        sv = idxb[pl.ds(CH + g * 16, 16)]
        md = jnp.where(dv == sv, trash, dv)
        mdb[sl] = md
        pkb[pl.ds(2 * CH + g * 16, 16)] = plsc.bitcast(md, jnp.float32)
        a0 = jnp.exp(gb[0][sl] - gb[4][sl])
        a1 = jnp.exp(gb[1][sl] - gb[5][sl])
        ex0b[sl] = a0
        ex1b[sl] = a1
        pkb[sl] = a0 * (gb[2][sl] + gb[6][sl])
        pkb[pl.ds(CH + g * 16, 16)] = a1 * (gb[3][sl] + gb[7][sl])

    def half(i, cur, nxt):
      @pl.when(i < nch)
      def _():
        wait_gathers(cur)

        @pl.when(i + 1 < nch)
        def _():
          issue_idx(i + 1, nxt)

        compute(cur)
        issue_outs(i)

        @pl.when(i + 1 < nch)
        def _():
          pltpu.make_async_copy(
              epk_ref.at[pl.ds(
                  pl.multiple_of(chunk_of(i + 1) * (2 * CH), 2 * CH),
                  2 * CH)], idxbs[nxt], sem_i).wait()
          issue_gathers(nxt)

    issue_idx(0, 0).wait()
    issue_gathers(0)

    def body(j, carry):
      half(2 * j, 0, 1)
      half(2 * j + 1, 1, 0)
      return carry

    lax.fori_loop(0, (base + 2) // 2, body, 0)
    plsc.subcore_barrier()
    for c in range(2):
      pltpu.sync_copy(dens[c].at[pl.ds(r0, rps)], stage)
      pltpu.sync_copy(stage, dpart_o.at[cid, c, pl.ds(r0, rps)])

  return k(epk, u0, u1, p0, p1, v0, v1, w0, w1)


def _sc_pass_b(pk, d0, d1, n_pad, e):
  nch_total = e // CH
  base, rem = nch_total // NW, nch_total % NW
  mesh = plsc.VectorSubcoreMesh(core_axis_name="c", subcore_axis_name="s")
  rps = n_pad // NS

  @functools.partial(
      pl.kernel,
      out_type=jax.ShapeDtypeStruct((NC, 2, n_pad), jnp.float32),
      mesh=mesh,
      compiler_params=pltpu.CompilerParams(needs_layout_passes=False),
      scratch_types=[
          [pltpu.VMEM((3 * CH,), jnp.float32) for _ in range(2)],
          [pltpu.VMEM((CH,), jnp.int32) for _ in range(2)],
          [[pltpu.VMEM((CH,), jnp.float32) for _ in range(2)]
           for _ in range(2)],
          pltpu.VMEM((CH,), jnp.float32),
          pltpu.VMEM((CH,), jnp.float32),
          pltpu.VMEM((rps,), jnp.float32),
          [pltpu.VMEM_SHARED((n_pad,), jnp.float32) for _ in range(2)],
          pltpu.SemaphoreType.DMA,
          pltpu.SemaphoreType.DMA,
          pltpu.SemaphoreType.DMA,
      ],
  )
  def k(pk_ref, d0_ref, d1_ref, opart_o,
        pkbs, mdbs, gbs, m0b, m1b, stage, outs, sem_i, sem_g, sem_o):
    cid = lax.axis_index("c")
    sid = lax.axis_index("s")
    wid = sid * NC + cid
    r0 = pl.multiple_of(sid * rps, 8)
    for g in range(rps // 16):
      stage[pl.ds(g * 16, 16)] = jnp.zeros((16,), jnp.float32)
    pltpu.sync_copy(stage, outs[0].at[pl.ds(r0, rps)])
    pltpu.sync_copy(stage, outs[1].at[pl.ds(r0, rps)])
    plsc.subcore_barrier()

    nch = jnp.where(wid < rem, base + 1, base)

    def chunk_of(i):
      return wid + i * NW

    def issue_pk(i, bank):
      i0 = pl.multiple_of(chunk_of(i) * (3 * CH), 3 * CH)
      return pltpu.async_copy(pk_ref.at[pl.ds(i0, 3 * CH)], pkbs[bank], sem_i)

    def wait_pk(i, bank):
      i0 = pl.multiple_of(chunk_of(i) * (3 * CH), 3 * CH)
      pltpu.make_async_copy(pk_ref.at[pl.ds(i0, 3 * CH)], pkbs[bank],
                            sem_i).wait()

    def extract_md(bank):
      for g in range(CH // 16):
        sl = pl.ds(g * 16, 16)
        mdbs[bank][sl] = plsc.bitcast(pkbs[bank][pl.ds(2 * CH + g * 16, 16)],
                                      jnp.int32)

    def issue_gathers(bank):
      pltpu.async_copy(d0_ref.at[mdbs[bank]], gbs[bank][0], sem_g)
      pltpu.async_copy(d1_ref.at[mdbs[bank]], gbs[bank][1], sem_g)

    def wait_gathers(bank):
      pltpu.make_async_copy(d0_ref.at[mdbs[bank]], gbs[bank][0], sem_g).wait()
      pltpu.make_async_copy(d1_ref.at[mdbs[bank]], gbs[bank][1], sem_g).wait()

    def issue_outs(bank):
      pltpu.sync_copy(m0b, outs[0].at[mdbs[bank]], add=True)
      pltpu.sync_copy(m1b, outs[1].at[mdbs[bank]], add=True)

    def compute(bank):
      for g in range(CH // 16):
        sl = pl.ds(g * 16, 16)
        m0b[sl] = pkbs[bank][sl] * gbs[bank][0][sl]
        m1b[sl] = pkbs[bank][pl.ds(CH + g * 16, 16)] * gbs[bank][1][sl]

    def half(i, cur, nxt):
      @pl.when(i < nch)
      def _():
        wait_gathers(cur)

        @pl.when(i + 1 < nch)
        def _():
          issue_pk(i + 1, nxt)

        compute(cur)
        issue_outs(cur)

        @pl.when(i + 1 < nch)
        def _():
          wait_pk(i + 1, nxt)
          extract_md(nxt)
          issue_gathers(nxt)

    issue_pk(0, 0).wait()
    extract_md(0)
    issue_gathers(0)

    def body(j, carry):
      half(2 * j, 0, 1)
      half(2 * j + 1, 1, 0)
      return carry

    lax.fori_loop(0, (base + 2) // 2, body, 0)
    plsc.subcore_barrier()
    for c in range(2):
      pltpu.sync_copy(outs[c].at[pl.ds(r0, rps)], stage)
      pltpu.sync_copy(stage, opart_o.at[cid, c, pl.ds(r0, rps)])

  return k(pk, d0, d1)


def kernel(x_pfc, edge_index, W_lin, W_src, W_dst, W_pos, b_pos):
  n = x_pfc.shape[0]
  e = edge_index.shape[1]
  n_pad = ((n + 1 + BLK - 1) // BLK) * BLK
  trash = jnp.int32(n)

  x_pad = jnp.pad(x_pfc, ((0, n_pad - n), (0, 0)))
  xt_aug = jnp.concatenate(
      [x_pad.T, jnp.ones((1, n_pad), jnp.float32)], axis=0)
  b2 = b_pos.reshape(2, 1)
  z2 = jnp.zeros((2, 1), jnp.float32)
  wd_t = jnp.concatenate([W_dst.T, b2], axis=1)
  ws_t = jnp.concatenate([W_src.T, z2], axis=1)
  wl_t = jnp.concatenate([W_lin.T, b2], axis=1)
  wp_t = W_pos.T
  epk = jnp.transpose(edge_index.reshape(2, e // CH, CH), (1, 0, 2)).reshape(-1)

  u0, u1, p0, p1, v0, v1, w0, w1, exs_t, ts_t = _tc_tables(
      xt_aug, wl_t, ws_t, wd_t, wp_t, n_pad)
  pk, dpart = _sc_pass_a(
      epk, u0, u1, p0, p1, v0, v1, w0, w1, n_pad, e, trash)
  d0, d1, selfmsg = _tc_dinv(dpart, exs_t, ts_t, n_pad)
  opart = _sc_pass_b(pk, d0, d1, n_pad, e)
  out_t = _tc_final(opart, selfmsg, n_pad)
  return out_t.T[:n]


# speedup vs baseline: 30.9697x; 1.3134x over previous
import functools

import jax
import jax.numpy as jnp
from jax import lax
from jax.experimental import pallas as pl
from jax.experimental.pallas import tpu as pltpu
from jax.experimental.pallas import tpu_sc as plsc

NC = 2
NS = 16
NW = NC * NS
CH = 128
BLK = 1024


def _tc_tables(xt_aug, wl_t, ws_t, wd_t, wp_t, n_pad):
  nblk = n_pad // BLK

  def body(x_ref, wl_ref, ws_ref, wd_ref, wp_ref,
           u0_r, u1_r, p0_r, p1_r, v0_r, v1_r, w0_r, w1_r, exs_r, ts_r):
    xt = x_ref[...]
    adt = jnp.dot(wd_ref[...], xt, preferred_element_type=jnp.float32, precision=lax.Precision.HIGHEST)
    avt = jnp.dot(ws_ref[...], xt, preferred_element_type=jnp.float32, precision=lax.Precision.HIGHEST)
    xvt = jnp.dot(wl_ref[...], xt, preferred_element_type=jnp.float32, precision=lax.Precision.HIGHEST)
    pt = jnp.dot(wp_ref[...], xt[:4, :], preferred_element_type=jnp.float32, precision=lax.Precision.HIGHEST)
    u0_r[...] = adt[0] + pt[0]
    u1_r[...] = adt[1] + pt[1]
    p0_r[...] = pt[0]
    p1_r[...] = pt[1]
    v0_r[...] = avt[0] + pt[0]
    v1_r[...] = avt[1] + pt[1]
    w0_r[...] = xvt[0] - pt[0]
    w1_r[...] = xvt[1] - pt[1]
    exs_r[...] = jnp.exp(adt - avt)
    ts_r[...] = xvt

  full = lambda s: pl.BlockSpec(s, lambda i: (0, 0))
  vec = pl.BlockSpec((BLK,), lambda i: (i,))
  row2 = pl.BlockSpec((2, BLK), lambda i: (0, i))
  vshape = jax.ShapeDtypeStruct((n_pad,), jnp.float32)
  return pl.pallas_call(
      body,
      grid=(nblk,),
      in_specs=[
          pl.BlockSpec((16, BLK), lambda i: (0, i)),
          full((2, 16)), full((2, 16)), full((2, 16)), full((2, 4)),
      ],
      out_specs=[vec] * 8 + [row2, row2],
      out_shape=[vshape] * 8 + [
          jax.ShapeDtypeStruct((2, n_pad), jnp.float32),
          jax.ShapeDtypeStruct((2, n_pad), jnp.float32),
      ],
  )(xt_aug, wl_t, ws_t, wd_t, wp_t)


def _tc_dinv(dpart, exs_t, ts_t, n_pad):
  nblk = n_pad // BLK

  def body(dp_ref, exs_ref, ts_ref, d0_ref, d1_ref, self_ref):
    dp = dp_ref[...]
    exs = exs_ref[...]
    dinv = 1.0 / (dp[0] + dp[1] + exs + 1e-16)
    d0_ref[...] = dinv[0]
    d1_ref[...] = dinv[1]
    self_ref[...] = exs * dinv * ts_ref[...]

  vec = pl.BlockSpec((BLK,), lambda i: (i,))
  row2 = pl.BlockSpec((2, BLK), lambda i: (0, i))
  return pl.pallas_call(
      body,
      grid=(nblk,),
      in_specs=[
          pl.BlockSpec((2, 2, BLK), lambda i: (0, 0, i)),
          row2, row2,
      ],
      out_specs=[vec, vec, row2],
      out_shape=[
          jax.ShapeDtypeStruct((n_pad,), jnp.float32),
          jax.ShapeDtypeStruct((n_pad,), jnp.float32),
          jax.ShapeDtypeStruct((2, n_pad), jnp.float32),
      ],
  )(dpart, exs_t, ts_t)


def _tc_final(opart, selfmsg, n_pad):
  nblk = n_pad // BLK

  def body(op_ref, sm_ref, out_ref):
    op = op_ref[...]
    out_ref[...] = op[0] + op[1] + sm_ref[...]

  row2 = pl.BlockSpec((2, BLK), lambda i: (0, i))
  return pl.pallas_call(
      body,
      grid=(nblk,),
      in_specs=[
          pl.BlockSpec((2, 2, BLK), lambda i: (0, 0, i)),
          row2,
      ],
      out_specs=row2,
      out_shape=jax.ShapeDtypeStruct((2, n_pad), jnp.float32),
  )(opart, selfmsg)


def _sc_pass_a(epk, u0, u1, p0, p1, v0, v1, w0, w1, n_pad, e, trash):
  nch_total = e // CH
  base, rem = nch_total // NW, nch_total % NW
  mesh = plsc.VectorSubcoreMesh(core_axis_name="c", subcore_axis_name="s")
  rps = n_pad // NS

  @functools.partial(
      pl.kernel,
      out_type=(
          jax.ShapeDtypeStruct((3 * e,), jnp.float32),
          jax.ShapeDtypeStruct((NC, 2, n_pad), jnp.float32),
      ),
      mesh=mesh,
      compiler_params=pltpu.CompilerParams(needs_layout_passes=False),
      scratch_types=[
          [pltpu.VMEM((2 * CH,), jnp.int32) for _ in range(2)],
          [[pltpu.VMEM((CH,), jnp.float32) for _ in range(8)]
           for _ in range(2)],
          pltpu.VMEM((CH,), jnp.int32),
          pltpu.VMEM((CH,), jnp.float32),
          pltpu.VMEM((CH,), jnp.float32),
          [pltpu.VMEM((3 * CH,), jnp.float32) for _ in range(2)],
          pltpu.VMEM((rps,), jnp.float32),
          [pltpu.VMEM_SHARED((n_pad,), jnp.float32) for _ in range(2)],
          pltpu.SemaphoreType.DMA,
          pltpu.SemaphoreType.DMA,
          pltpu.SemaphoreType.DMA,
      ],
  )
  def k(epk_ref, u0_ref, u1_ref, p0_ref, p1_ref, v0_ref, v1_ref, w0_ref,
        w1_ref, pk_o, dpart_o,
        idxbs, gbs, mdb, ex0b, ex1b, pkbs, stage, dens, sem_i, sem_g, sem_o):
    cid = lax.axis_index("c")
    sid = lax.axis_index("s")
    wid = sid * NC + cid
    r0 = pl.multiple_of(sid * rps, 8)
    for g in range(rps // 16):
      stage[pl.ds(g * 16, 16)] = jnp.zeros((16,), jnp.float32)
    pltpu.sync_copy(stage, dens[0].at[pl.ds(r0, rps)])
    pltpu.sync_copy(stage, dens[1].at[pl.ds(r0, rps)])
    plsc.subcore_barrier()

    tabs = (u0_ref, u1_ref, p0_ref, p1_ref, v0_ref, v1_ref, w0_ref, w1_ref)
    nch = jnp.where(wid < rem, base + 1, base)

    def chunk_of(i):
      return wid + i * NW

    def issue_idx(i, bank):
      i0 = pl.multiple_of(chunk_of(i) * (2 * CH), 2 * CH)
      return pltpu.async_copy(epk_ref.at[pl.ds(i0, 2 * CH)], idxbs[bank],
                              sem_i)

    def wait_idx(i, bank):
      i0 = pl.multiple_of(chunk_of(i) * (2 * CH), 2 * CH)
      pltpu.make_async_copy(epk_ref.at[pl.ds(i0, 2 * CH)], idxbs[bank],
                            sem_i).wait()

    def issue_gathers(bank):
      for t in range(8):
        sl = pl.ds(0, CH) if t < 4 else pl.ds(CH, CH)
        pltpu.async_copy(tabs[t].at[idxbs[bank].at[sl]], gbs[bank][t], sem_g)

    def wait_gathers(bank):
      for t in range(8):
        sl = pl.ds(0, CH) if t < 4 else pl.ds(CH, CH)
        pltpu.make_async_copy(tabs[t].at[idxbs[bank].at[sl]], gbs[bank][t],
                              sem_g).wait()

    def wait_pk(i, bank):
      e0p = pl.multiple_of(chunk_of(i) * (3 * CH), 3 * CH)
      pltpu.make_async_copy(pkbs[bank], pk_o.at[pl.ds(e0p, 3 * CH)],
                            sem_o).wait()

    def drain_pk(i):
      @pl.when(i % 2 == 0)
      def _():
        wait_pk(i, 0)

      @pl.when(i % 2 == 1)
      def _():
        wait_pk(i, 1)

    def compute(bank):
      idxb = idxbs[bank]
      gb = gbs[bank]
      pkb = pkbs[bank]
      for g in range(CH // 16):
        sl = pl.ds(g * 16, 16)
        dv = idxb[sl]
        sv = idxb[pl.ds(CH + g * 16, 16)]
        md = jnp.where(dv == sv, trash, dv)
        mdb[sl] = md
        pkb[pl.ds(2 * CH + g * 16, 16)] = plsc.bitcast(md, jnp.float32)
        a0 = jnp.exp(gb[0][sl] - gb[4][sl])
        a1 = jnp.exp(gb[1][sl] - gb[5][sl])
        ex0b[sl] = a0
        ex1b[sl] = a1
        pkb[sl] = a0 * (gb[2][sl] + gb[6][sl])
        pkb[pl.ds(CH + g * 16, 16)] = a1 * (gb[3][sl] + gb[7][sl])

    def half(i, cur, nxt):
      @pl.when(i < nch)
      def _():
        wait_gathers(cur)

        @pl.when(i + 1 < nch)
        def _():
          wait_idx(i + 1, nxt)
          issue_gathers(nxt)

        @pl.when(i >= 2)
        def _():
          wait_pk(i - 2, cur)

        compute(cur)

        @pl.when(i + 2 < nch)
        def _():
          issue_idx(i + 2, cur)

        pltpu.sync_copy(ex0b, dens[0].at[mdb], add=True)
        pltpu.sync_copy(ex1b, dens[1].at[mdb], add=True)
        e0p = pl.multiple_of(chunk_of(i) * (3 * CH), 3 * CH)
        pltpu.async_copy(pkbs[cur], pk_o.at[pl.ds(e0p, 3 * CH)], sem_o)

    issue_idx(0, 0).wait()
    issue_gathers(0)

    @pl.when(1 < nch)
    def _():
      issue_idx(1, 1)

    def body(j, carry):
      half(2 * j, 0, 1)
      half(2 * j + 1, 1, 0)
      return carry

    lax.fori_loop(0, (base + 2) // 2, body, 0)

    @pl.when(nch >= 2)
    def _():
      drain_pk(nch - 2)

    drain_pk(nch - 1)
    plsc.subcore_barrier()
    for c in range(2):
      pltpu.sync_copy(dens[c].at[pl.ds(r0, rps)], stage)
      pltpu.sync_copy(stage, dpart_o.at[cid, c, pl.ds(r0, rps)])

  return k(epk, u0, u1, p0, p1, v0, v1, w0, w1)


def _sc_pass_b(pk, d0, d1, n_pad, e):
  nch_total = e // CH
  base, rem = nch_total // NW, nch_total % NW
  mesh = plsc.VectorSubcoreMesh(core_axis_name="c", subcore_axis_name="s")
  rps = n_pad // NS

  @functools.partial(
      pl.kernel,
      out_type=jax.ShapeDtypeStruct((NC, 2, n_pad), jnp.float32),
      mesh=mesh,
      compiler_params=pltpu.CompilerParams(needs_layout_passes=False),
      scratch_types=[
          [pltpu.VMEM((3 * CH,), jnp.float32) for _ in range(2)],
          [pltpu.VMEM((CH,), jnp.int32) for _ in range(2)],
          [[pltpu.VMEM((CH,), jnp.float32) for _ in range(2)]
           for _ in range(2)],
          pltpu.VMEM((CH,), jnp.float32),
          pltpu.VMEM((CH,), jnp.float32),
          pltpu.VMEM((rps,), jnp.float32),
          [pltpu.VMEM_SHARED((n_pad,), jnp.float32) for _ in range(2)],
          pltpu.SemaphoreType.DMA,
          pltpu.SemaphoreType.DMA,
          pltpu.SemaphoreType.DMA,
      ],
  )
  def k(pk_ref, d0_ref, d1_ref, opart_o,
        pkbs, mdbs, gbs, m0b, m1b, stage, outs, sem_i, sem_g, sem_o):
    cid = lax.axis_index("c")
    sid = lax.axis_index("s")
    wid = sid * NC + cid
    r0 = pl.multiple_of(sid * rps, 8)
    for g in range(rps // 16):
      stage[pl.ds(g * 16, 16)] = jnp.zeros((16,), jnp.float32)
    pltpu.sync_copy(stage, outs[0].at[pl.ds(r0, rps)])
    pltpu.sync_copy(stage, outs[1].at[pl.ds(r0, rps)])
    plsc.subcore_barrier()

    nch = jnp.where(wid < rem, base + 1, base)

    def chunk_of(i):
      return wid + i * NW

    def issue_pk(i, bank):
      i0 = pl.multiple_of(chunk_of(i) * (3 * CH), 3 * CH)
      return pltpu.async_copy(pk_ref.at[pl.ds(i0, 3 * CH)], pkbs[bank], sem_i)

    def wait_pk(i, bank):
      i0 = pl.multiple_of(chunk_of(i) * (3 * CH), 3 * CH)
      pltpu.make_async_copy(pk_ref.at[pl.ds(i0, 3 * CH)], pkbs[bank],
                            sem_i).wait()

    def extract_md(bank):
      for g in range(CH // 16):
        sl = pl.ds(g * 16, 16)
        mdbs[bank][sl] = plsc.bitcast(pkbs[bank][pl.ds(2 * CH + g * 16, 16)],
                                      jnp.int32)

    def issue_gathers(bank):
      pltpu.async_copy(d0_ref.at[mdbs[bank]], gbs[bank][0], sem_g)
      pltpu.async_copy(d1_ref.at[mdbs[bank]], gbs[bank][1], sem_g)

    def wait_gathers(bank):
      pltpu.make_async_copy(d0_ref.at[mdbs[bank]], gbs[bank][0], sem_g).wait()
      pltpu.make_async_copy(d1_ref.at[mdbs[bank]], gbs[bank][1], sem_g).wait()

    def issue_outs(bank):
      pltpu.sync_copy(m0b, outs[0].at[mdbs[bank]], add=True)
      pltpu.sync_copy(m1b, outs[1].at[mdbs[bank]], add=True)

    def compute(bank):
      for g in range(CH // 16):
        sl = pl.ds(g * 16, 16)
        m0b[sl] = pkbs[bank][sl] * gbs[bank][0][sl]
        m1b[sl] = pkbs[bank][pl.ds(CH + g * 16, 16)] * gbs[bank][1][sl]

    def half(i, cur, nxt):
      @pl.when(i < nch)
      def _():
        wait_gathers(cur)

        @pl.when(i + 1 < nch)
        def _():
          wait_pk(i + 1, nxt)
          extract_md(nxt)
          issue_gathers(nxt)

        compute(cur)

        @pl.when(i + 2 < nch)
        def _():
          issue_pk(i + 2, cur)

        issue_outs(cur)

    issue_pk(0, 0).wait()
    extract_md(0)
    issue_gathers(0)

    @pl.when(1 < nch)
    def _():
      issue_pk(1, 1)

    def body(j, carry):
      half(2 * j, 0, 1)
      half(2 * j + 1, 1, 0)
      return carry

    lax.fori_loop(0, (base + 2) // 2, body, 0)
    plsc.subcore_barrier()
    for c in range(2):
      pltpu.sync_copy(outs[c].at[pl.ds(r0, rps)], stage)
      pltpu.sync_copy(stage, opart_o.at[cid, c, pl.ds(r0, rps)])

  return k(pk, d0, d1)


def kernel(x_pfc, edge_index, W_lin, W_src, W_dst, W_pos, b_pos):
  n = x_pfc.shape[0]
  e = edge_index.shape[1]
  n_pad = ((n + 1 + BLK - 1) // BLK) * BLK
  trash = jnp.int32(n)

  x_pad = jnp.pad(x_pfc, ((0, n_pad - n), (0, 0)))
  xt_aug = jnp.concatenate(
      [x_pad.T, jnp.ones((1, n_pad), jnp.float32)], axis=0)
  b2 = b_pos.reshape(2, 1)
  z2 = jnp.zeros((2, 1), jnp.float32)
  wd_t = jnp.concatenate([W_dst.T, b2], axis=1)
  ws_t = jnp.concatenate([W_src.T, z2], axis=1)
  wl_t = jnp.concatenate([W_lin.T, b2], axis=1)
  wp_t = W_pos.T
  epk = jnp.transpose(edge_index.reshape(2, e // CH, CH), (1, 0, 2)).reshape(-1)

  u0, u1, p0, p1, v0, v1, w0, w1, exs_t, ts_t = _tc_tables(
      xt_aug, wl_t, ws_t, wd_t, wp_t, n_pad)
  pk, dpart = _sc_pass_a(
      epk, u0, u1, p0, p1, v0, v1, w0, w1, n_pad, e, trash)
  d0, d1, selfmsg = _tc_dinv(dpart, exs_t, ts_t, n_pad)
  opart = _sc_pass_b(pk, d0, d1, n_pad, e)
  out_t = _tc_final(opart, selfmsg, n_pad)
  return out_t.T[:n]


# speedup vs baseline: 43.3150x; 1.3986x over previous
import functools

import jax
import jax.numpy as jnp
from jax import lax
from jax.experimental import pallas as pl
from jax.experimental.pallas import tpu as pltpu
from jax.experimental.pallas import tpu_sc as plsc

NC = 2
NS = 16
NW = NC * NS
CH = 512
BLK = 1024


def _tc_tables(xt_aug, wl_t, ws_t, wd_t, wp_t, n_pad):
  nblk = n_pad // BLK

  def body(x_ref, wl_ref, ws_ref, wd_ref, wp_ref,
           u0_r, u1_r, p0_r, p1_r, v0_r, v1_r, w0_r, w1_r, exs_r, ts_r):
    xt = x_ref[...]
    adt = jnp.dot(wd_ref[...], xt, preferred_element_type=jnp.float32, precision=lax.Precision.HIGHEST)
    avt = jnp.dot(ws_ref[...], xt, preferred_element_type=jnp.float32, precision=lax.Precision.HIGHEST)
    xvt = jnp.dot(wl_ref[...], xt, preferred_element_type=jnp.float32, precision=lax.Precision.HIGHEST)
    pt = jnp.dot(wp_ref[...], xt[:4, :], preferred_element_type=jnp.float32, precision=lax.Precision.HIGHEST)
    u0_r[...] = adt[0] + pt[0]
    u1_r[...] = adt[1] + pt[1]
    p0_r[...] = pt[0]
    p1_r[...] = pt[1]
    v0_r[...] = avt[0] + pt[0]
    v1_r[...] = avt[1] + pt[1]
    w0_r[...] = xvt[0] - pt[0]
    w1_r[...] = xvt[1] - pt[1]
    exs_r[...] = jnp.exp(adt - avt)
    ts_r[...] = xvt

  full = lambda s: pl.BlockSpec(s, lambda i: (0, 0))
  vec = pl.BlockSpec((BLK,), lambda i: (i,))
  row2 = pl.BlockSpec((2, BLK), lambda i: (0, i))
  vshape = jax.ShapeDtypeStruct((n_pad,), jnp.float32)
  return pl.pallas_call(
      body,
      grid=(nblk,),
      in_specs=[
          pl.BlockSpec((16, BLK), lambda i: (0, i)),
          full((2, 16)), full((2, 16)), full((2, 16)), full((2, 4)),
      ],
      out_specs=[vec] * 8 + [row2, row2],
      out_shape=[vshape] * 8 + [
          jax.ShapeDtypeStruct((2, n_pad), jnp.float32),
          jax.ShapeDtypeStruct((2, n_pad), jnp.float32),
      ],
  )(xt_aug, wl_t, ws_t, wd_t, wp_t)


def _tc_dinv(dpart, exs_t, ts_t, n_pad):
  nblk = n_pad // BLK

  def body(dp_ref, exs_ref, ts_ref, d0_ref, d1_ref, self_ref):
    dp = dp_ref[...]
    exs = exs_ref[...]
    dinv = 1.0 / (dp[0] + dp[1] + exs + 1e-16)
    d0_ref[...] = dinv[0]
    d1_ref[...] = dinv[1]
    self_ref[...] = exs * dinv * ts_ref[...]

  vec = pl.BlockSpec((BLK,), lambda i: (i,))
  row2 = pl.BlockSpec((2, BLK), lambda i: (0, i))
  return pl.pallas_call(
      body,
      grid=(nblk,),
      in_specs=[
          pl.BlockSpec((2, 2, BLK), lambda i: (0, 0, i)),
          row2, row2,
      ],
      out_specs=[vec, vec, row2],
      out_shape=[
          jax.ShapeDtypeStruct((n_pad,), jnp.float32),
          jax.ShapeDtypeStruct((n_pad,), jnp.float32),
          jax.ShapeDtypeStruct((2, n_pad), jnp.float32),
      ],
  )(dpart, exs_t, ts_t)


def _tc_final(opart, selfmsg, n_pad):
  nblk = n_pad // BLK

  def body(op_ref, sm_ref, out_ref):
    op = op_ref[...]
    out_ref[...] = op[0] + op[1] + sm_ref[...]

  row2 = pl.BlockSpec((2, BLK), lambda i: (0, i))
  return pl.pallas_call(
      body,
      grid=(nblk,),
      in_specs=[
          pl.BlockSpec((2, 2, BLK), lambda i: (0, 0, i)),
          row2,
      ],
      out_specs=row2,
      out_shape=jax.ShapeDtypeStruct((2, n_pad), jnp.float32),
  )(opart, selfmsg)


def _sc_pass_a(epk, u0, u1, p0, p1, v0, v1, w0, w1, n_pad, e, trash):
  nch_total = e // CH
  base, rem = nch_total // NW, nch_total % NW
  mesh = plsc.VectorSubcoreMesh(core_axis_name="c", subcore_axis_name="s")
  rps = n_pad // NS

  @functools.partial(
      pl.kernel,
      out_type=(
          jax.ShapeDtypeStruct((3 * e,), jnp.float32),
          jax.ShapeDtypeStruct((NC, 2, n_pad), jnp.float32),
      ),
      mesh=mesh,
      compiler_params=pltpu.CompilerParams(needs_layout_passes=False),
      scratch_types=[
          [pltpu.VMEM((2 * CH,), jnp.int32) for _ in range(2)],
          [[pltpu.VMEM((CH,), jnp.float32) for _ in range(8)]
           for _ in range(2)],
          pltpu.VMEM((CH,), jnp.int32),
          pltpu.VMEM((CH,), jnp.float32),
          pltpu.VMEM((CH,), jnp.float32),
          [pltpu.VMEM((3 * CH,), jnp.float32) for _ in range(2)],
          pltpu.VMEM((rps,), jnp.float32),
          [pltpu.VMEM_SHARED((n_pad,), jnp.float32) for _ in range(2)],
          pltpu.SemaphoreType.DMA,
          pltpu.SemaphoreType.DMA,
          pltpu.SemaphoreType.DMA,
      ],
  )
  def k(epk_ref, u0_ref, u1_ref, p0_ref, p1_ref, v0_ref, v1_ref, w0_ref,
        w1_ref, pk_o, dpart_o,
        idxbs, gbs, mdb, ex0b, ex1b, pkbs, stage, dens, sem_i, sem_g, sem_o):
    cid = lax.axis_index("c")
    sid = lax.axis_index("s")
    wid = sid * NC + cid
    r0 = pl.multiple_of(sid * rps, 8)
    for g in range(rps // 16):
      stage[pl.ds(g * 16, 16)] = jnp.zeros((16,), jnp.float32)
    pltpu.sync_copy(stage, dens[0].at[pl.ds(r0, rps)])
    pltpu.sync_copy(stage, dens[1].at[pl.ds(r0, rps)])
    plsc.subcore_barrier()

    tabs = (u0_ref, u1_ref, p0_ref, p1_ref, v0_ref, v1_ref, w0_ref, w1_ref)
    nch = jnp.where(wid < rem, base + 1, base)

    def chunk_of(i):
      return wid + i * NW

    def issue_idx(i, bank):
      i0 = pl.multiple_of(chunk_of(i) * (2 * CH), 2 * CH)
      return pltpu.async_copy(epk_ref.at[pl.ds(i0, 2 * CH)], idxbs[bank],
                              sem_i)

    def wait_idx(i, bank):
      i0 = pl.multiple_of(chunk_of(i) * (2 * CH), 2 * CH)
      pltpu.make_async_copy(epk_ref.at[pl.ds(i0, 2 * CH)], idxbs[bank],
                            sem_i).wait()

    def issue_gathers(bank):
      for t in range(8):
        sl = pl.ds(0, CH) if t < 4 else pl.ds(CH, CH)
        pltpu.async_copy(tabs[t].at[idxbs[bank].at[sl]], gbs[bank][t], sem_g)

    def wait_gathers(bank):
      for t in range(8):
        sl = pl.ds(0, CH) if t < 4 else pl.ds(CH, CH)
        pltpu.make_async_copy(tabs[t].at[idxbs[bank].at[sl]], gbs[bank][t],
                              sem_g).wait()

    def wait_pk(i, bank):
      e0p = pl.multiple_of(chunk_of(i) * (3 * CH), 3 * CH)
      pltpu.make_async_copy(pkbs[bank], pk_o.at[pl.ds(e0p, 3 * CH)],
                            sem_o).wait()

    def drain_pk(i):
      @pl.when(i % 2 == 0)
      def _():
        wait_pk(i, 0)

      @pl.when(i % 2 == 1)
      def _():
        wait_pk(i, 1)

    def compute(bank):
      idxb = idxbs[bank]
      gb = gbs[bank]
      pkb = pkbs[bank]
      for g in range(CH // 16):
        sl = pl.ds(g * 16, 16)
        dv = idxb[sl]
        sv = idxb[pl.ds(CH + g * 16, 16)]
        md = jnp.where(dv == sv, trash, dv)
        mdb[sl] = md
        pkb[pl.ds(2 * CH + g * 16, 16)] = plsc.bitcast(md, jnp.float32)
        a0 = jnp.exp(gb[0][sl] - gb[4][sl])
        a1 = jnp.exp(gb[1][sl] - gb[5][sl])
        ex0b[sl] = a0
        ex1b[sl] = a1
        pkb[sl] = a0 * (gb[2][sl] + gb[6][sl])
        pkb[pl.ds(CH + g * 16, 16)] = a1 * (gb[3][sl] + gb[7][sl])

    def half(i, cur, nxt):
      @pl.when(i < nch)
      def _():
        wait_gathers(cur)

        @pl.when(i + 1 < nch)
        def _():
          wait_idx(i + 1, nxt)
          issue_gathers(nxt)

        @pl.when(i >= 2)
        def _():
          wait_pk(i - 2, cur)

        compute(cur)

        @pl.when(i + 2 < nch)
        def _():
          issue_idx(i + 2, cur)

        pltpu.sync_copy(ex0b, dens[0].at[mdb], add=True)
        pltpu.sync_copy(ex1b, dens[1].at[mdb], add=True)
        e0p = pl.multiple_of(chunk_of(i) * (3 * CH), 3 * CH)
        pltpu.async_copy(pkbs[cur], pk_o.at[pl.ds(e0p, 3 * CH)], sem_o)

    issue_idx(0, 0).wait()
    issue_gathers(0)

    @pl.when(1 < nch)
    def _():
      issue_idx(1, 1)

    def body(j, carry):
      half(2 * j, 0, 1)
      half(2 * j + 1, 1, 0)
      return carry

    lax.fori_loop(0, (base + 2) // 2, body, 0)

    @pl.when(nch >= 2)
    def _():
      drain_pk(nch - 2)

    drain_pk(nch - 1)
    plsc.subcore_barrier()
    for c in range(2):
      pltpu.sync_copy(dens[c].at[pl.ds(r0, rps)], stage)
      pltpu.sync_copy(stage, dpart_o.at[cid, c, pl.ds(r0, rps)])

  return k(epk, u0, u1, p0, p1, v0, v1, w0, w1)


def _sc_pass_b(pk, d0, d1, n_pad, e):
  nch_total = e // CH
  base, rem = nch_total // NW, nch_total % NW
  mesh = plsc.VectorSubcoreMesh(core_axis_name="c", subcore_axis_name="s")
  rps = n_pad // NS

  @functools.partial(
      pl.kernel,
      out_type=jax.ShapeDtypeStruct((NC, 2, n_pad), jnp.float32),
      mesh=mesh,
      compiler_params=pltpu.CompilerParams(needs_layout_passes=False),
      scratch_types=[
          [pltpu.VMEM((3 * CH,), jnp.float32) for _ in range(2)],
          [pltpu.VMEM((CH,), jnp.int32) for _ in range(2)],
          [[pltpu.VMEM((CH,), jnp.float32) for _ in range(2)]
           for _ in range(2)],
          pltpu.VMEM((CH,), jnp.float32),
          pltpu.VMEM((CH,), jnp.float32),
          pltpu.VMEM((rps,), jnp.float32),
          [pltpu.VMEM_SHARED((n_pad,), jnp.float32) for _ in range(2)],
          pltpu.SemaphoreType.DMA,
          pltpu.SemaphoreType.DMA,
          pltpu.SemaphoreType.DMA,
      ],
  )
  def k(pk_ref, d0_ref, d1_ref, opart_o,
        pkbs, mdbs, gbs, m0b, m1b, stage, outs, sem_i, sem_g, sem_o):
    cid = lax.axis_index("c")
    sid = lax.axis_index("s")
    wid = sid * NC + cid
    r0 = pl.multiple_of(sid * rps, 8)
    for g in range(rps // 16):
      stage[pl.ds(g * 16, 16)] = jnp.zeros((16,), jnp.float32)
    pltpu.sync_copy(stage, outs[0].at[pl.ds(r0, rps)])
    pltpu.sync_copy(stage, outs[1].at[pl.ds(r0, rps)])
    plsc.subcore_barrier()

    nch = jnp.where(wid < rem, base + 1, base)

    def chunk_of(i):
      return wid + i * NW

    def issue_pk(i, bank):
      i0 = pl.multiple_of(chunk_of(i) * (3 * CH), 3 * CH)
      return pltpu.async_copy(pk_ref.at[pl.ds(i0, 3 * CH)], pkbs[bank], sem_i)

    def wait_pk(i, bank):
      i0 = pl.multiple_of(chunk_of(i) * (3 * CH), 3 * CH)
      pltpu.make_async_copy(pk_ref.at[pl.ds(i0, 3 * CH)], pkbs[bank],
                            sem_i).wait()

    def extract_md(bank):
      for g in range(CH // 16):
        sl = pl.ds(g * 16, 16)
        mdbs[bank][sl] = plsc.bitcast(pkbs[bank][pl.ds(2 * CH + g * 16, 16)],
                                      jnp.int32)

    def issue_gathers(bank):
      pltpu.async_copy(d0_ref.at[mdbs[bank]], gbs[bank][0], sem_g)
      pltpu.async_copy(d1_ref.at[mdbs[bank]], gbs[bank][1], sem_g)

    def wait_gathers(bank):
      pltpu.make_async_copy(d0_ref.at[mdbs[bank]], gbs[bank][0], sem_g).wait()
      pltpu.make_async_copy(d1_ref.at[mdbs[bank]], gbs[bank][1], sem_g).wait()

    def issue_outs(bank):
      pltpu.sync_copy(m0b, outs[0].at[mdbs[bank]], add=True)
      pltpu.sync_copy(m1b, outs[1].at[mdbs[bank]], add=True)

    def compute(bank):
      for g in range(CH // 16):
        sl = pl.ds(g * 16, 16)
        m0b[sl] = pkbs[bank][sl] * gbs[bank][0][sl]
        m1b[sl] = pkbs[bank][pl.ds(CH + g * 16, 16)] * gbs[bank][1][sl]

    def half(i, cur, nxt):
      @pl.when(i < nch)
      def _():
        wait_gathers(cur)

        @pl.when(i + 1 < nch)
        def _():
          wait_pk(i + 1, nxt)
          extract_md(nxt)
          issue_gathers(nxt)

        compute(cur)

        @pl.when(i + 2 < nch)
        def _():
          issue_pk(i + 2, cur)

        issue_outs(cur)

    issue_pk(0, 0).wait()
    extract_md(0)
    issue_gathers(0)

    @pl.when(1 < nch)
    def _():
      issue_pk(1, 1)

    def body(j, carry):
      half(2 * j, 0, 1)
      half(2 * j + 1, 1, 0)
      return carry

    lax.fori_loop(0, (base + 2) // 2, body, 0)
    plsc.subcore_barrier()
    for c in range(2):
      pltpu.sync_copy(outs[c].at[pl.ds(r0, rps)], stage)
      pltpu.sync_copy(stage, opart_o.at[cid, c, pl.ds(r0, rps)])

  return k(pk, d0, d1)


def kernel(x_pfc, edge_index, W_lin, W_src, W_dst, W_pos, b_pos):
  n = x_pfc.shape[0]
  e = edge_index.shape[1]
  n_pad = ((n + 1 + BLK - 1) // BLK) * BLK
  trash = jnp.int32(n)

  x_pad = jnp.pad(x_pfc, ((0, n_pad - n), (0, 0)))
  xt_aug = jnp.concatenate(
      [x_pad.T, jnp.ones((1, n_pad), jnp.float32)], axis=0)
  b2 = b_pos.reshape(2, 1)
  z2 = jnp.zeros((2, 1), jnp.float32)
  wd_t = jnp.concatenate([W_dst.T, b2], axis=1)
  ws_t = jnp.concatenate([W_src.T, z2], axis=1)
  wl_t = jnp.concatenate([W_lin.T, b2], axis=1)
  wp_t = W_pos.T
  epk = jnp.transpose(edge_index.reshape(2, e // CH, CH), (1, 0, 2)).reshape(-1)

  u0, u1, p0, p1, v0, v1, w0, w1, exs_t, ts_t = _tc_tables(
      xt_aug, wl_t, ws_t, wd_t, wp_t, n_pad)
  pk, dpart = _sc_pass_a(
      epk, u0, u1, p0, p1, v0, v1, w0, w1, n_pad, e, trash)
  d0, d1, selfmsg = _tc_dinv(dpart, exs_t, ts_t, n_pad)
  opart = _sc_pass_b(pk, d0, d1, n_pad, e)
  out_t = _tc_final(opart, selfmsg, n_pad)
  return out_t.T[:n]


# speedup vs baseline: 46.7220x; 1.0787x over previous
import functools

import jax
import jax.numpy as jnp
from jax import lax
from jax.experimental import pallas as pl
from jax.experimental.pallas import tpu as pltpu
from jax.experimental.pallas import tpu_sc as plsc

NC = 2
NS = 16
NW = NC * NS
CH = 1024
BLK = 1024


def _tc_tables(xt_aug, wl_t, ws_t, wd_t, wp_t, n_pad):
  nblk = n_pad // BLK

  def body(x_ref, wl_ref, ws_ref, wd_ref, wp_ref,
           u0_r, u1_r, p0_r, p1_r, v0_r, v1_r, w0_r, w1_r, exs_r, ts_r):
    xt = x_ref[...]
    adt = jnp.dot(wd_ref[...], xt, preferred_element_type=jnp.float32, precision=lax.Precision.HIGHEST)
    avt = jnp.dot(ws_ref[...], xt, preferred_element_type=jnp.float32, precision=lax.Precision.HIGHEST)
    xvt = jnp.dot(wl_ref[...], xt, preferred_element_type=jnp.float32, precision=lax.Precision.HIGHEST)
    pt = jnp.dot(wp_ref[...], xt[:4, :], preferred_element_type=jnp.float32, precision=lax.Precision.HIGHEST)
    u0_r[...] = adt[0] + pt[0]
    u1_r[...] = adt[1] + pt[1]
    p0_r[...] = pt[0]
    p1_r[...] = pt[1]
    v0_r[...] = avt[0] + pt[0]
    v1_r[...] = avt[1] + pt[1]
    w0_r[...] = xvt[0] - pt[0]
    w1_r[...] = xvt[1] - pt[1]
    exs_r[...] = jnp.exp(adt - avt)
    ts_r[...] = xvt

  full = lambda s: pl.BlockSpec(s, lambda i: (0, 0))
  vec = pl.BlockSpec((BLK,), lambda i: (i,))
  row2 = pl.BlockSpec((2, BLK), lambda i: (0, i))
  vshape = jax.ShapeDtypeStruct((n_pad,), jnp.float32)
  return pl.pallas_call(
      body,
      grid=(nblk,),
      in_specs=[
          pl.BlockSpec((16, BLK), lambda i: (0, i)),
          full((2, 16)), full((2, 16)), full((2, 16)), full((2, 4)),
      ],
      out_specs=[vec] * 8 + [row2, row2],
      out_shape=[vshape] * 8 + [
          jax.ShapeDtypeStruct((2, n_pad), jnp.float32),
          jax.ShapeDtypeStruct((2, n_pad), jnp.float32),
      ],
  )(xt_aug, wl_t, ws_t, wd_t, wp_t)


def _tc_dinv(dpart, exs_t, ts_t, n_pad):
  nblk = n_pad // BLK

  def body(dp_ref, exs_ref, ts_ref, d0_ref, d1_ref, self_ref):
    dp = dp_ref[...]
    exs = exs_ref[...]
    dinv = 1.0 / (dp[0] + dp[1] + exs + 1e-16)
    d0_ref[...] = dinv[0]
    d1_ref[...] = dinv[1]
    self_ref[...] = exs * dinv * ts_ref[...]

  vec = pl.BlockSpec((BLK,), lambda i: (i,))
  row2 = pl.BlockSpec((2, BLK), lambda i: (0, i))
  return pl.pallas_call(
      body,
      grid=(nblk,),
      in_specs=[
          pl.BlockSpec((2, 2, BLK), lambda i: (0, 0, i)),
          row2, row2,
      ],
      out_specs=[vec, vec, row2],
      out_shape=[
          jax.ShapeDtypeStruct((n_pad,), jnp.float32),
          jax.ShapeDtypeStruct((n_pad,), jnp.float32),
          jax.ShapeDtypeStruct((2, n_pad), jnp.float32),
      ],
  )(dpart, exs_t, ts_t)


def _tc_final(opart, selfmsg, n_pad):
  nblk = n_pad // BLK

  def body(op_ref, sm_ref, out_ref):
    op = op_ref[...]
    out_ref[...] = op[0] + op[1] + sm_ref[...]

  row2 = pl.BlockSpec((2, BLK), lambda i: (0, i))
  return pl.pallas_call(
      body,
      grid=(nblk,),
      in_specs=[
          pl.BlockSpec((2, 2, BLK), lambda i: (0, 0, i)),
          row2,
      ],
      out_specs=row2,
      out_shape=jax.ShapeDtypeStruct((2, n_pad), jnp.float32),
  )(opart, selfmsg)


def _sc_pass_a(epk, u0, u1, p0, p1, v0, v1, w0, w1, n_pad, e, trash):
  nch_total = e // CH
  base, rem = nch_total // NW, nch_total % NW
  mesh = plsc.VectorSubcoreMesh(core_axis_name="c", subcore_axis_name="s")
  rps = n_pad // NS

  @functools.partial(
      pl.kernel,
      out_type=(
          jax.ShapeDtypeStruct((3 * e,), jnp.float32),
          jax.ShapeDtypeStruct((NC, 2, n_pad), jnp.float32),
      ),
      mesh=mesh,
      compiler_params=pltpu.CompilerParams(needs_layout_passes=False),
      scratch_types=[
          [pltpu.VMEM((2 * CH,), jnp.int32) for _ in range(2)],
          [[pltpu.VMEM((CH,), jnp.float32) for _ in range(8)]
           for _ in range(2)],
          pltpu.VMEM((CH,), jnp.int32),
          pltpu.VMEM((CH,), jnp.float32),
          pltpu.VMEM((CH,), jnp.float32),
          [pltpu.VMEM((3 * CH,), jnp.float32) for _ in range(2)],
          pltpu.VMEM((rps,), jnp.float32),
          [pltpu.VMEM_SHARED((n_pad,), jnp.float32) for _ in range(2)],
          pltpu.SemaphoreType.DMA,
          pltpu.SemaphoreType.DMA,
          pltpu.SemaphoreType.DMA,
      ],
  )
  def k(epk_ref, u0_ref, u1_ref, p0_ref, p1_ref, v0_ref, v1_ref, w0_ref,
        w1_ref, pk_o, dpart_o,
        idxbs, gbs, mdb, ex0b, ex1b, pkbs, stage, dens, sem_i, sem_g, sem_o):
    cid = lax.axis_index("c")
    sid = lax.axis_index("s")
    wid = sid * NC + cid
    r0 = pl.multiple_of(sid * rps, 8)
    for g in range(rps // 16):
      stage[pl.ds(g * 16, 16)] = jnp.zeros((16,), jnp.float32)
    pltpu.sync_copy(stage, dens[0].at[pl.ds(r0, rps)])
    pltpu.sync_copy(stage, dens[1].at[pl.ds(r0, rps)])
    plsc.subcore_barrier()

    tabs = (u0_ref, u1_ref, p0_ref, p1_ref, v0_ref, v1_ref, w0_ref, w1_ref)
    nch = jnp.where(wid < rem, base + 1, base)

    def chunk_of(i):
      return wid + i * NW

    def issue_idx(i, bank):
      i0 = pl.multiple_of(chunk_of(i) * (2 * CH), 2 * CH)
      return pltpu.async_copy(epk_ref.at[pl.ds(i0, 2 * CH)], idxbs[bank],
                              sem_i)

    def wait_idx(i, bank):
      i0 = pl.multiple_of(chunk_of(i) * (2 * CH), 2 * CH)
      pltpu.make_async_copy(epk_ref.at[pl.ds(i0, 2 * CH)], idxbs[bank],
                            sem_i).wait()

    def issue_gathers(bank):
      for t in range(8):
        sl = pl.ds(0, CH) if t < 4 else pl.ds(CH, CH)
        pltpu.async_copy(tabs[t].at[idxbs[bank].at[sl]], gbs[bank][t], sem_g)

    def wait_gathers(bank):
      for t in range(8):
        sl = pl.ds(0, CH) if t < 4 else pl.ds(CH, CH)
        pltpu.make_async_copy(tabs[t].at[idxbs[bank].at[sl]], gbs[bank][t],
                              sem_g).wait()

    def wait_pk(i, bank):
      e0p = pl.multiple_of(chunk_of(i) * (3 * CH), 3 * CH)
      pltpu.make_async_copy(pkbs[bank], pk_o.at[pl.ds(e0p, 3 * CH)],
                            sem_o).wait()

    def drain_pk(i):
      @pl.when(i % 2 == 0)
      def _():
        wait_pk(i, 0)

      @pl.when(i % 2 == 1)
      def _():
        wait_pk(i, 1)

    def compute(bank):
      idxb = idxbs[bank]
      gb = gbs[bank]
      pkb = pkbs[bank]
      for g in range(CH // 16):
        sl = pl.ds(g * 16, 16)
        dv = idxb[sl]
        sv = idxb[pl.ds(CH + g * 16, 16)]
        md = jnp.where(dv == sv, trash, dv)
        mdb[sl] = md
        pkb[pl.ds(2 * CH + g * 16, 16)] = plsc.bitcast(md, jnp.float32)
        a0 = jnp.exp(gb[0][sl] - gb[4][sl])
        a1 = jnp.exp(gb[1][sl] - gb[5][sl])
        ex0b[sl] = a0
        ex1b[sl] = a1
        pkb[sl] = a0 * (gb[2][sl] + gb[6][sl])
        pkb[pl.ds(CH + g * 16, 16)] = a1 * (gb[3][sl] + gb[7][sl])

    def half(i, cur, nxt):
      @pl.when(i < nch)
      def _():
        wait_gathers(cur)

        @pl.when(i + 1 < nch)
        def _():
          wait_idx(i + 1, nxt)
          issue_gathers(nxt)

        @pl.when(i >= 2)
        def _():
          wait_pk(i - 2, cur)

        compute(cur)

        @pl.when(i + 2 < nch)
        def _():
          issue_idx(i + 2, cur)

        pltpu.sync_copy(ex0b, dens[0].at[mdb], add=True)
        pltpu.sync_copy(ex1b, dens[1].at[mdb], add=True)
        e0p = pl.multiple_of(chunk_of(i) * (3 * CH), 3 * CH)
        pltpu.async_copy(pkbs[cur], pk_o.at[pl.ds(e0p, 3 * CH)], sem_o)

    issue_idx(0, 0).wait()
    issue_gathers(0)

    @pl.when(1 < nch)
    def _():
      issue_idx(1, 1)

    def body(j, carry):
      half(2 * j, 0, 1)
      half(2 * j + 1, 1, 0)
      return carry

    lax.fori_loop(0, (base + 2) // 2, body, 0)

    @pl.when(nch >= 2)
    def _():
      drain_pk(nch - 2)

    drain_pk(nch - 1)
    plsc.subcore_barrier()
    for c in range(2):
      pltpu.sync_copy(dens[c].at[pl.ds(r0, rps)], stage)
      pltpu.sync_copy(stage, dpart_o.at[cid, c, pl.ds(r0, rps)])

  return k(epk, u0, u1, p0, p1, v0, v1, w0, w1)


def _sc_pass_b(pk, d0, d1, n_pad, e):
  nch_total = e // CH
  base, rem = nch_total // NW, nch_total % NW
  mesh = plsc.VectorSubcoreMesh(core_axis_name="c", subcore_axis_name="s")
  rps = n_pad // NS

  @functools.partial(
      pl.kernel,
      out_type=jax.ShapeDtypeStruct((NC, 2, n_pad), jnp.float32),
      mesh=mesh,
      compiler_params=pltpu.CompilerParams(needs_layout_passes=False),
      scratch_types=[
          [pltpu.VMEM((3 * CH,), jnp.float32) for _ in range(2)],
          [pltpu.VMEM((CH,), jnp.int32) for _ in range(2)],
          [[pltpu.VMEM((CH,), jnp.float32) for _ in range(2)]
           for _ in range(2)],
          pltpu.VMEM((CH,), jnp.float32),
          pltpu.VMEM((CH,), jnp.float32),
          pltpu.VMEM((rps,), jnp.float32),
          [pltpu.VMEM_SHARED((n_pad,), jnp.float32) for _ in range(2)],
          pltpu.SemaphoreType.DMA,
          pltpu.SemaphoreType.DMA,
          pltpu.SemaphoreType.DMA,
      ],
  )
  def k(pk_ref, d0_ref, d1_ref, opart_o,
        pkbs, mdbs, gbs, m0b, m1b, stage, outs, sem_i, sem_g, sem_o):
    cid = lax.axis_index("c")
    sid = lax.axis_index("s")
    wid = sid * NC + cid
    r0 = pl.multiple_of(sid * rps, 8)
    for g in range(rps // 16):
      stage[pl.ds(g * 16, 16)] = jnp.zeros((16,), jnp.float32)
    pltpu.sync_copy(stage, outs[0].at[pl.ds(r0, rps)])
    pltpu.sync_copy(stage, outs[1].at[pl.ds(r0, rps)])
    plsc.subcore_barrier()

    nch = jnp.where(wid < rem, base + 1, base)

    def chunk_of(i):
      return wid + i * NW

    def issue_pk(i, bank):
      i0 = pl.multiple_of(chunk_of(i) * (3 * CH), 3 * CH)
      return pltpu.async_copy(pk_ref.at[pl.ds(i0, 3 * CH)], pkbs[bank], sem_i)

    def wait_pk(i, bank):
      i0 = pl.multiple_of(chunk_of(i) * (3 * CH), 3 * CH)
      pltpu.make_async_copy(pk_ref.at[pl.ds(i0, 3 * CH)], pkbs[bank],
                            sem_i).wait()

    def extract_md(bank):
      for g in range(CH // 16):
        sl = pl.ds(g * 16, 16)
        mdbs[bank][sl] = plsc.bitcast(pkbs[bank][pl.ds(2 * CH + g * 16, 16)],
                                      jnp.int32)

    def issue_gathers(bank):
      pltpu.async_copy(d0_ref.at[mdbs[bank]], gbs[bank][0], sem_g)
      pltpu.async_copy(d1_ref.at[mdbs[bank]], gbs[bank][1], sem_g)

    def wait_gathers(bank):
      pltpu.make_async_copy(d0_ref.at[mdbs[bank]], gbs[bank][0], sem_g).wait()
      pltpu.make_async_copy(d1_ref.at[mdbs[bank]], gbs[bank][1], sem_g).wait()

    def issue_outs(bank):
      pltpu.sync_copy(m0b, outs[0].at[mdbs[bank]], add=True)
      pltpu.sync_copy(m1b, outs[1].at[mdbs[bank]], add=True)

    def compute(bank):
      for g in range(CH // 16):
        sl = pl.ds(g * 16, 16)
        m0b[sl] = pkbs[bank][sl] * gbs[bank][0][sl]
        m1b[sl] = pkbs[bank][pl.ds(CH + g * 16, 16)] * gbs[bank][1][sl]

    def half(i, cur, nxt):
      @pl.when(i < nch)
      def _():
        wait_gathers(cur)

        @pl.when(i + 1 < nch)
        def _():
          wait_pk(i + 1, nxt)
          extract_md(nxt)
          issue_gathers(nxt)

        compute(cur)

        @pl.when(i + 2 < nch)
        def _():
          issue_pk(i + 2, cur)

        issue_outs(cur)

    issue_pk(0, 0).wait()
    extract_md(0)
    issue_gathers(0)

    @pl.when(1 < nch)
    def _():
      issue_pk(1, 1)

    def body(j, carry):
      half(2 * j, 0, 1)
      half(2 * j + 1, 1, 0)
      return carry

    lax.fori_loop(0, (base + 2) // 2, body, 0)
    plsc.subcore_barrier()
    for c in range(2):
      pltpu.sync_copy(outs[c].at[pl.ds(r0, rps)], stage)
      pltpu.sync_copy(stage, opart_o.at[cid, c, pl.ds(r0, rps)])

  return k(pk, d0, d1)


def kernel(x_pfc, edge_index, W_lin, W_src, W_dst, W_pos, b_pos):
  n = x_pfc.shape[0]
  e = edge_index.shape[1]
  n_pad = ((n + 1 + BLK - 1) // BLK) * BLK
  trash = jnp.int32(n)

  x_pad = jnp.pad(x_pfc, ((0, n_pad - n), (0, 0)))
  xt_aug = jnp.concatenate(
      [x_pad.T, jnp.ones((1, n_pad), jnp.float32)], axis=0)
  b2 = b_pos.reshape(2, 1)
  z2 = jnp.zeros((2, 1), jnp.float32)
  wd_t = jnp.concatenate([W_dst.T, b2], axis=1)
  ws_t = jnp.concatenate([W_src.T, z2], axis=1)
  wl_t = jnp.concatenate([W_lin.T, b2], axis=1)
  wp_t = W_pos.T
  epk = jnp.transpose(edge_index.reshape(2, e // CH, CH), (1, 0, 2)).reshape(-1)

  u0, u1, p0, p1, v0, v1, w0, w1, exs_t, ts_t = _tc_tables(
      xt_aug, wl_t, ws_t, wd_t, wp_t, n_pad)
  pk, dpart = _sc_pass_a(
      epk, u0, u1, p0, p1, v0, v1, w0, w1, n_pad, e, trash)
  d0, d1, selfmsg = _tc_dinv(dpart, exs_t, ts_t, n_pad)
  opart = _sc_pass_b(pk, d0, d1, n_pad, e)
  out_t = _tc_final(opart, selfmsg, n_pad)
  return out_t.T[:n]
